# baseline (speedup 1.0000x reference)
import functools

import jax
import jax.numpy as jnp
from jax import lax
from jax.experimental import pallas as pl
from jax.experimental.pallas import tpu as pltpu
from jax.experimental.pallas import tpu_sc as plsc

N = 10000
F = 256
OUT = 256
NH = 4
E = 160000
HALF = 128
LANES = 16

RB = 400
GRID = N // RB

NC = 2
NS = 16
EPT = E // NS
BB = 80
NBATCH = EPT // BB
NSLOT = 4
R0 = 624
TAIL = N - R0 * NS
ZB = 16


def _mm_body(x_ref, w_ref, le_ref, fw_ref, g_ref, acc_ref):
    i = pl.program_id(0)
    p = lax.dot_general(x_ref[...], w_ref[...], (((1,), (0,)), ((), ())),
                        preferred_element_type=jnp.float32)
    le = p[:, :OUT]
    le_ref[...] = le
    for h in range(NH):
        base = OUT + h * OUT
        for s in range(NC):
            fw_ref[h, s] = p[:, base + s * HALF:base + (s + 1) * HALF]
    fwsum = (p[:, OUT:2 * OUT] + p[:, 2 * OUT:3 * OUT]
             + p[:, 3 * OUT:4 * OUT] + p[:, 4 * OUT:5 * OUT])
    contrib = lax.dot_general(le, fwsum, (((0,), (0,)), ((), ())),
                              preferred_element_type=jnp.float32)

    @pl.when(i == 0)
    def _():
        acc_ref[...] = jnp.zeros_like(acc_ref)

    acc_ref[...] += contrib

    @pl.when(i == pl.num_programs(0) - 1)
    def _():
        g_ref[...] = acc_ref[...]


_mm_call = pl.pallas_call(
    _mm_body,
    grid=(GRID,),
    in_specs=[
        pl.BlockSpec((RB, F), lambda i: (i, 0)),
        pl.BlockSpec((F, (NH + 1) * OUT), lambda i: (0, 0)),
    ],
    out_specs=[
        pl.BlockSpec((RB, OUT), lambda i: (i, 0)),
        pl.BlockSpec((NH, NC, RB, HALF), lambda i: (0, 0, i, 0)),
        pl.BlockSpec((OUT, OUT), lambda i: (0, 0)),
    ],
    out_shape=[
        jax.ShapeDtypeStruct((N, OUT), jnp.float32),
        jax.ShapeDtypeStruct((NH, NC, N, HALF), jnp.float32),
        jax.ShapeDtypeStruct((OUT, OUT), jnp.float32),
    ],
    scratch_shapes=[pltpu.VMEM((OUT, OUT), jnp.float32)],
)


def _sc_spmm_body(fw_hbm, rows_hbm, cols_hbm, vals_hbm, out_hbm,
                  g0, g1, g2, g3, cb0, cb1, cb2, cb3, vb0, vb1, vb2, vb3,
                  ib0, ib1, ib2, ib3, rb0, rb1, rb2, rb3, zbuf_v,
                  acc_sh, gs0, gs1, gs2, gs3, ss0, ss1, ss2, ss3,
                  ms0, ms1, ms2, ms3, rs0, rs1, rs2, rs3):
    c = lax.axis_index("c")
    s = lax.axis_index("s")
    gath = (g0, g1, g2, g3)
    colsb = (cb0, cb1, cb2, cb3)
    valsb = (vb0, vb1, vb2, vb3)
    idxb = (ib0, ib1, ib2, ib3)
    rbs = (rb0, rb1, rb2, rb3)
    gsem = (gs0, gs1, gs2, gs3)
    ssem = (ss0, ss1, ss2, ss3)
    msem = (ms0, ms1, ms2, ms3)
    rsem = (rs0, rs1, rs2, rs3)

    for r in range(ZB):
        for j in range(HALF // LANES):
            zbuf_v[r, pl.ds(j * LANES, LANES)] = jnp.zeros((LANES,),
                                                           jnp.float32)

    def zcopy(k, carry):
        pltpu.sync_copy(zbuf_v, acc_sh.at[pl.ds(s * R0 + k * ZB, ZB)])
        return carry

    lax.fori_loop(0, R0 // ZB, zcopy, 0)

    @pl.when(s == NS - 1)
    def _():
        pltpu.sync_copy(zbuf_v, acc_sh.at[pl.ds(R0 * NS, TAIL)])

    plsc.subcore_barrier()

    def issue_meta(slot, hb, b):
        pltpu.async_copy(cols_hbm.at[pl.ds(hb + b * BB, BB)], colsb[slot],
                         msem[slot])
        pltpu.async_copy(vals_hbm.at[pl.ds(hb + b * BB, BB)], valsb[slot],
                         msem[slot])

    def wait_meta(slot, hb):
        pltpu.make_async_copy(cols_hbm.at[pl.ds(hb, BB)], colsb[slot],
                              msem[slot]).wait()
        pltpu.make_async_copy(vals_hbm.at[pl.ds(hb, BB)], valsb[slot],
                              msem[slot]).wait()

    def mkidx(slot, off_vec):
        for j in range(BB // LANES):
            sl = pl.ds(j * LANES, LANES)
            idxb[slot][sl] = colsb[slot][sl] + off_vec

    def issue_gather_rows(slot, hb, b):
        pltpu.async_copy(fw_hbm.at[idxb[slot]], gath[slot], gsem[slot])
        pltpu.async_copy(rows_hbm.at[pl.ds(hb + b * BB, BB)], rbs[slot],
                         rsem[slot])

    def wait_gather(slot):
        pltpu.make_async_copy(fw_hbm.at[idxb[slot]], gath[slot],
                              gsem[slot]).wait()

    def wait_rows(slot, hb):
        pltpu.make_async_copy(rows_hbm.at[pl.ds(hb, BB)], rbs[slot],
                              rsem[slot]).wait()

    def wait_scatter(slot):
        pltpu.make_async_copy(gath[slot], acc_sh.at[rbs[slot]],
                              ssem[slot]).wait()

    def issue_scatter(slot):
        pltpu.async_copy(gath[slot], acc_sh.at[rbs[slot]], ssem[slot],
                         add=True)

    def scale(slot):
        def grp(g, carry):
            vv = valsb[slot][pl.ds(g * LANES, LANES)]
            for l in range(LANES):
                r = g * LANES + l
                v = vv[l]
                for j in range(HALF // LANES):
                    sl = pl.ds(j * LANES, LANES)
                    gath[slot][r, sl] = gath[slot][r, sl] * v
            return carry

        lax.fori_loop(0, BB // LANES, grp, 0)

    def hop_body(h, carry):
        hb = h * E + s * EPT
        off_vec = jnp.zeros((LANES,), jnp.int32) + (h * NC + c) * N

        issue_meta(0, hb, jnp.int32(0))
        wait_meta(0, hb)
        mkidx(0, off_vec)
        issue_gather_rows(0, hb, jnp.int32(0))
        issue_meta(1, hb, jnp.int32(1))

        def step(p, k):
            b = NSLOT * p + k
            nk = (k + 1) % NSLOT
            wait_gather(k)
            wait_meta(nk, hb)
            mkidx(nk, off_vec)
            if k == NSLOT - 1:
                wait_scatter(nk)
            else:
                @pl.when(p > 0)
                def _():
                    wait_scatter(nk)
            issue_gather_rows(nk, hb, b + 1)
            scale(k)
            mk = (k + 2) % NSLOT
            if k == NSLOT - 1:
                @pl.when(p < NBATCH // NSLOT - 1)
                def _():
                    issue_meta(mk, hb, b + 2)
            else:
                issue_meta(mk, hb, b + 2)
            wait_rows(k, hb)
            issue_scatter(k)

        def quad(p, cy):
            for k in range(NSLOT):
                step(p, k)
            return cy

        lax.fori_loop(0, NBATCH // NSLOT, quad, 0)

        wait_gather(0)
        scale(0)
        wait_rows(0, hb)
        issue_scatter(0)
        for k in range(NSLOT):
            wait_scatter((1 + k) % NSLOT)
        return carry

    lax.fori_loop(0, NH, hop_body, 0)

    plsc.subcore_barrier()
    pltpu.sync_copy(acc_sh.at[pl.ds(s * R0, R0)],
                    out_hbm.at[c, pl.ds(s * R0, R0)])

    @pl.when(s == NS - 1)
    def _():
        pltpu.sync_copy(acc_sh.at[pl.ds(R0 * NS, TAIL)],
                        out_hbm.at[c, pl.ds(R0 * NS, TAIL)])


_sc_call = functools.partial(
    pl.kernel,
    mesh=plsc.VectorSubcoreMesh(core_axis_name="c", subcore_axis_name="s"),
    out_type=jax.ShapeDtypeStruct((NC, N, HALF), jnp.float32),
    scratch_types=(
        [pltpu.VMEM((BB, HALF), jnp.float32) for _ in range(NSLOT)]
        + [pltpu.VMEM((BB,), jnp.int32) for _ in range(NSLOT)]
        + [pltpu.VMEM((BB,), jnp.float32) for _ in range(NSLOT)]
        + [pltpu.VMEM((BB,), jnp.int32) for _ in range(NSLOT)]
        + [pltpu.VMEM((BB,), jnp.int32) for _ in range(NSLOT)]
        + [pltpu.VMEM((ZB, HALF), jnp.float32)]
        + [pltpu.VMEM_SHARED((N, HALF), jnp.float32)]
        + [pltpu.SemaphoreType.DMA for _ in range(4 * NSLOT)]
    ),
)(_sc_spmm_body)


def _out_body(alpha_ref, st_ref, le_ref, g_ref, o_ref):
    a = alpha_ref[0, 0]
    lg = lax.dot_general(le_ref[...], g_ref[...], (((1,), (0,)), ((), ())),
                         preferred_element_type=jnp.float32)
    o_ref[:, :HALF] = jnp.maximum(
        st_ref[0].astype(jnp.float32) + a * lg[:, :HALF], 0.0)
    o_ref[:, HALF:] = jnp.maximum(
        st_ref[1].astype(jnp.float32) + a * lg[:, HALF:], 0.0)


_out_call = pl.pallas_call(
    _out_body,
    grid=(GRID,),
    in_specs=[
        pl.BlockSpec(memory_space=pltpu.SMEM),
        pl.BlockSpec((NC, RB, HALF), lambda i: (0, i, 0)),
        pl.BlockSpec((RB, OUT), lambda i: (i, 0)),
        pl.BlockSpec((OUT, OUT), lambda i: (0, 0)),
    ],
    out_specs=pl.BlockSpec((RB, OUT), lambda i: (i, 0)),
    out_shape=jax.ShapeDtypeStruct((N, OUT), jnp.float32),
)


def kernel(X, adj_edge_index, adj_values, W_embed, W_kernels, alpha):
    w_cat = jnp.concatenate(
        [W_embed] + [W_kernels[h] for h in range(NH)], axis=1)
    le, fw, g = _mm_call(X, w_cat)
    fw_flat = fw.reshape(NH * NC * N, HALF)
    rows_flat = adj_edge_index[:, 0, :].reshape(-1)
    cols_flat = adj_edge_index[:, 1, :].reshape(-1)
    vals_flat = adj_values.reshape(-1)
    structural = _sc_call(fw_flat, rows_flat, cols_flat, vals_flat)
    alpha_arr = jnp.reshape(alpha, (1, 1)).astype(jnp.float32)
    return _out_call(alpha_arr, structural, le, g)


# speedup vs baseline: 1.2951x; 1.2951x over previous
import functools

import jax
import jax.numpy as jnp
from jax import lax
from jax.experimental import pallas as pl
from jax.experimental.pallas import tpu as pltpu
from jax.experimental.pallas import tpu_sc as plsc

N = 10000
F = 256
OUT = 256
NH = 4
E = 160000
HALF = 128
LANES = 16

RB = 400
GRID = N // RB

NC = 2
NS = 16
EPT = E // NS
BB = 80
NBATCH = EPT // BB
NSLOT = 4
R0 = 624
TAIL = N - R0 * NS
ZB = 16


def _mm_body(x_ref, w_ref, le_ref, fw_ref, g_ref, acc_ref):
    i = pl.program_id(0)
    p = lax.dot_general(x_ref[...], w_ref[...], (((1,), (0,)), ((), ())),
                        preferred_element_type=jnp.float32)
    le = p[:, :OUT]
    le_ref[...] = le
    for h in range(NH):
        base = OUT + h * OUT
        for s in range(NC):
            fw_ref[h, s] = p[:, base + s * HALF:base + (s + 1) * HALF]
    fwsum = (p[:, OUT:2 * OUT] + p[:, 2 * OUT:3 * OUT]
             + p[:, 3 * OUT:4 * OUT] + p[:, 4 * OUT:5 * OUT])
    contrib = lax.dot_general(le, fwsum, (((0,), (0,)), ((), ())),
                              preferred_element_type=jnp.float32)

    @pl.when(i == 0)
    def _():
        acc_ref[...] = jnp.zeros_like(acc_ref)

    acc_ref[...] += contrib

    @pl.when(i == pl.num_programs(0) - 1)
    def _():
        g_ref[...] = acc_ref[...]


_mm_call = pl.pallas_call(
    _mm_body,
    grid=(GRID,),
    in_specs=[
        pl.BlockSpec((RB, F), lambda i: (i, 0)),
        pl.BlockSpec((F, (NH + 1) * OUT), lambda i: (0, 0)),
    ],
    out_specs=[
        pl.BlockSpec((RB, OUT), lambda i: (i, 0)),
        pl.BlockSpec((NH, NC, RB, HALF), lambda i: (0, 0, i, 0)),
        pl.BlockSpec((OUT, OUT), lambda i: (0, 0)),
    ],
    out_shape=[
        jax.ShapeDtypeStruct((N, OUT), jnp.float32),
        jax.ShapeDtypeStruct((NH, NC, N, HALF), jnp.float32),
        jax.ShapeDtypeStruct((OUT, OUT), jnp.float32),
    ],
    scratch_shapes=[pltpu.VMEM((OUT, OUT), jnp.float32)],
)


def _sc_spmm_body(fw_hbm, rows_hbm, cols_hbm, vals_hbm, out_hbm,
                  g0, g1, g2, g3, cb0, cb1, cb2, cb3, vb0, vb1, vb2, vb3,
                  ib0, ib1, ib2, ib3, rb0, rb1, rb2, rb3, zbuf_v,
                  acc_sh, gs0, gs1, gs2, gs3, ss0, ss1, ss2, ss3,
                  ms0, ms1, ms2, ms3, rs0, rs1, rs2, rs3):
    c = lax.axis_index("c")
    s = lax.axis_index("s")
    gath = (g0, g1, g2, g3)
    colsb = (cb0, cb1, cb2, cb3)
    valsb = (vb0, vb1, vb2, vb3)
    idxb = (ib0, ib1, ib2, ib3)
    rbs = (rb0, rb1, rb2, rb3)
    gsem = (gs0, gs1, gs2, gs3)
    ssem = (ss0, ss1, ss2, ss3)
    msem = (ms0, ms1, ms2, ms3)
    rsem = (rs0, rs1, rs2, rs3)

    for r in range(ZB):
        for j in range(HALF // LANES):
            zbuf_v[r, pl.ds(j * LANES, LANES)] = jnp.zeros((LANES,),
                                                           jnp.float32)

    def zcopy(k, carry):
        pltpu.sync_copy(zbuf_v, acc_sh.at[pl.ds(s * R0 + k * ZB, ZB)])
        return carry

    lax.fori_loop(0, R0 // ZB, zcopy, 0)

    @pl.when(s == NS - 1)
    def _():
        pltpu.sync_copy(zbuf_v, acc_sh.at[pl.ds(R0 * NS, TAIL)])

    plsc.subcore_barrier()

    def issue_meta(slot, hb, b):
        pltpu.async_copy(cols_hbm.at[pl.ds(hb + b * BB, BB)], colsb[slot],
                         msem[slot])
        pltpu.async_copy(vals_hbm.at[pl.ds(hb + b * BB, BB)], valsb[slot],
                         msem[slot])

    def wait_meta(slot, hb):
        pltpu.make_async_copy(cols_hbm.at[pl.ds(hb, BB)], colsb[slot],
                              msem[slot]).wait()
        pltpu.make_async_copy(vals_hbm.at[pl.ds(hb, BB)], valsb[slot],
                              msem[slot]).wait()

    def mkidx(slot, off_vec):
        for j in range(BB // LANES):
            sl = pl.ds(j * LANES, LANES)
            idxb[slot][sl] = colsb[slot][sl] + off_vec

    def issue_gather_rows(slot, hb, b):
        pltpu.async_copy(fw_hbm.at[idxb[slot]], gath[slot], gsem[slot])
        pltpu.async_copy(rows_hbm.at[pl.ds(hb + b * BB, BB)], rbs[slot],
                         rsem[slot])

    def wait_gather(slot):
        pltpu.make_async_copy(fw_hbm.at[idxb[slot]], gath[slot],
                              gsem[slot]).wait()

    def wait_rows(slot, hb):
        pltpu.make_async_copy(rows_hbm.at[pl.ds(hb, BB)], rbs[slot],
                              rsem[slot]).wait()

    def wait_scatter(slot):
        pltpu.make_async_copy(gath[slot], acc_sh.at[rbs[slot]],
                              ssem[slot]).wait()

    def issue_scatter(slot):
        pltpu.async_copy(gath[slot], acc_sh.at[rbs[slot]], ssem[slot],
                         add=True)

    def scale(slot):
        def grp(g, carry):
            vv = valsb[slot][pl.ds(g * LANES, LANES)]
            for l in range(LANES):
                r = g * LANES + l
                v = vv[l]
                for j in range(HALF // LANES):
                    sl = pl.ds(j * LANES, LANES)
                    gath[slot][r, sl] = gath[slot][r, sl] * v
            return carry

        lax.fori_loop(0, BB // LANES, grp, 0)

    def hop_body(h, carry):
        hb = h * E + s * EPT
        off_vec = jnp.zeros((LANES,), jnp.int32) + (h * NC + c) * N

        issue_meta(0, hb, jnp.int32(0))
        issue_meta(1, hb, jnp.int32(1))
        wait_meta(0, hb)
        mkidx(0, off_vec)
        issue_gather_rows(0, hb, jnp.int32(0))
        wait_meta(1, hb)
        mkidx(1, off_vec)
        issue_gather_rows(1, hb, jnp.int32(1))
        issue_meta(2, hb, jnp.int32(2))
        issue_meta(3, hb, jnp.int32(3))

        NQ = NBATCH // NSLOT

        def step(p, k):
            b = NSLOT * p + k
            g2 = (k + 2) % NSLOT
            wait_gather(k)
            if k < 2:
                @pl.when(p > 0)
                def _():
                    wait_scatter(g2)
            else:
                wait_scatter(g2)
            if k == NSLOT - 1:
                @pl.when(p < NQ - 1)
                def _():
                    wait_meta(g2, hb)
                    mkidx(g2, off_vec)
                    issue_gather_rows(g2, hb, b + 2)
            else:
                wait_meta(g2, hb)
                mkidx(g2, off_vec)
                issue_gather_rows(g2, hb, b + 2)
            scale(k)
            if k == 0:
                issue_meta(k, hb, b + NSLOT)
            else:
                @pl.when(p < NQ - 1)
                def _():
                    issue_meta(k, hb, b + NSLOT)
            wait_rows(k, hb)
            issue_scatter(k)

        def quad(p, cy):
            for k in range(NSLOT):
                step(p, k)
            return cy

        lax.fori_loop(0, NQ, quad, 0)

        wait_gather(0)
        scale(0)
        wait_rows(0, hb)
        issue_scatter(0)
        wait_scatter(2)
        wait_scatter(3)
        wait_scatter(0)
        return carry

    lax.fori_loop(0, NH, hop_body, 0)

    plsc.subcore_barrier()
    pltpu.sync_copy(acc_sh.at[pl.ds(s * R0, R0)],
                    out_hbm.at[c, pl.ds(s * R0, R0)])

    @pl.when(s == NS - 1)
    def _():
        pltpu.sync_copy(acc_sh.at[pl.ds(R0 * NS, TAIL)],
                        out_hbm.at[c, pl.ds(R0 * NS, TAIL)])


_sc_call = functools.partial(
    pl.kernel,
    mesh=plsc.VectorSubcoreMesh(core_axis_name="c", subcore_axis_name="s"),
    out_type=jax.ShapeDtypeStruct((NC, N, HALF), jnp.float32),
    scratch_types=(
        [pltpu.VMEM((BB, HALF), jnp.float32) for _ in range(NSLOT)]
        + [pltpu.VMEM((BB,), jnp.int32) for _ in range(NSLOT)]
        + [pltpu.VMEM((BB,), jnp.float32) for _ in range(NSLOT)]
        + [pltpu.VMEM((BB,), jnp.int32) for _ in range(NSLOT)]
        + [pltpu.VMEM((BB,), jnp.int32) for _ in range(NSLOT)]
        + [pltpu.VMEM((ZB, HALF), jnp.float32)]
        + [pltpu.VMEM_SHARED((N, HALF), jnp.float32)]
        + [pltpu.SemaphoreType.DMA for _ in range(4 * NSLOT)]
    ),
)(_sc_spmm_body)


def _out_body(alpha_ref, st_ref, le_ref, g_ref, o_ref):
    a = alpha_ref[0, 0]
    lg = lax.dot_general(le_ref[...], g_ref[...], (((1,), (0,)), ((), ())),
                         preferred_element_type=jnp.float32)
    o_ref[:, :HALF] = jnp.maximum(
        st_ref[0].astype(jnp.float32) + a * lg[:, :HALF], 0.0)
    o_ref[:, HALF:] = jnp.maximum(
        st_ref[1].astype(jnp.float32) + a * lg[:, HALF:], 0.0)


_out_call = pl.pallas_call(
    _out_body,
    grid=(GRID,),
    in_specs=[
        pl.BlockSpec(memory_space=pltpu.SMEM),
        pl.BlockSpec((NC, RB, HALF), lambda i: (0, i, 0)),
        pl.BlockSpec((RB, OUT), lambda i: (i, 0)),
        pl.BlockSpec((OUT, OUT), lambda i: (0, 0)),
    ],
    out_specs=pl.BlockSpec((RB, OUT), lambda i: (i, 0)),
    out_shape=jax.ShapeDtypeStruct((N, OUT), jnp.float32),
)


def kernel(X, adj_edge_index, adj_values, W_embed, W_kernels, alpha):
    w_cat = jnp.concatenate(
        [W_embed] + [W_kernels[h] for h in range(NH)], axis=1)
    le, fw, g = _mm_call(X, w_cat)
    fw_flat = fw.reshape(NH * NC * N, HALF)
    rows_flat = adj_edge_index[:, 0, :].reshape(-1)
    cols_flat = adj_edge_index[:, 1, :].reshape(-1)
    vals_flat = adj_values.reshape(-1)
    structural = _sc_call(fw_flat, rows_flat, cols_flat, vals_flat)
    alpha_arr = jnp.reshape(alpha, (1, 1)).astype(jnp.float32)
    return _out_call(alpha_arr, structural, le, g)


# speedup vs baseline: 1.2970x; 1.0015x over previous
import functools

import jax
import jax.numpy as jnp
from jax import lax
from jax.experimental import pallas as pl
from jax.experimental.pallas import tpu as pltpu
from jax.experimental.pallas import tpu_sc as plsc

N = 10000
F = 256
OUT = 256
NH = 4
E = 160000
HALF = 128
LANES = 16

RB = 400
GRID = N // RB

NC = 2
NS = 16
EPT = E // NS
BB = 80
NBATCH = EPT // BB
NSLOT = 4
R0 = 624
TAIL = N - R0 * NS
ZB = 16


def _mm_body(x_ref, w_ref, le_ref, fw_ref, g_ref, acc_ref):
    i = pl.program_id(0)
    p = lax.dot_general(x_ref[...], w_ref[...], (((1,), (0,)), ((), ())),
                        preferred_element_type=jnp.float32)
    le = p[:, :OUT]
    le_ref[...] = le
    for h in range(NH):
        base = OUT + h * OUT
        for s in range(NC):
            fw_ref[h, s] = p[:, base + s * HALF:base + (s + 1) * HALF]
    fwsum = (p[:, OUT:2 * OUT] + p[:, 2 * OUT:3 * OUT]
             + p[:, 3 * OUT:4 * OUT] + p[:, 4 * OUT:5 * OUT])
    contrib = lax.dot_general(le, fwsum, (((0,), (0,)), ((), ())),
                              preferred_element_type=jnp.float32)

    @pl.when(i == 0)
    def _():
        acc_ref[...] = jnp.zeros_like(acc_ref)

    acc_ref[...] += contrib

    @pl.when(i == pl.num_programs(0) - 1)
    def _():
        g_ref[...] = acc_ref[...]


_mm_call = pl.pallas_call(
    _mm_body,
    grid=(GRID,),
    in_specs=[
        pl.BlockSpec((RB, F), lambda i: (i, 0)),
        pl.BlockSpec((F, (NH + 1) * OUT), lambda i: (0, 0)),
    ],
    out_specs=[
        pl.BlockSpec((RB, OUT), lambda i: (i, 0)),
        pl.BlockSpec((NH, NC, RB, HALF), lambda i: (0, 0, i, 0)),
        pl.BlockSpec((OUT, OUT), lambda i: (0, 0)),
    ],
    out_shape=[
        jax.ShapeDtypeStruct((N, OUT), jnp.float32),
        jax.ShapeDtypeStruct((NH, NC, N, HALF), jnp.float32),
        jax.ShapeDtypeStruct((OUT, OUT), jnp.float32),
    ],
    scratch_shapes=[pltpu.VMEM((OUT, OUT), jnp.float32)],
)


def _sc_spmm_body(fw_hbm, rows_hbm, cols_hbm, vals_hbm, out_hbm,
                  g0, g1, g2, g3, cb0, cb1, cb2, cb3, vb0, vb1, vb2, vb3,
                  ib0, ib1, ib2, ib3, rb0, rb1, rb2, rb3, zbuf_v,
                  acc_sh, gs0, gs1, gs2, gs3, ss0, ss1, ss2, ss3,
                  ms0, ms1, ms2, ms3, rs0, rs1, rs2, rs3):
    c = lax.axis_index("c")
    s = lax.axis_index("s")
    gath = (g0, g1, g2, g3)
    colsb = (cb0, cb1, cb2, cb3)
    valsb = (vb0, vb1, vb2, vb3)
    idxb = (ib0, ib1, ib2, ib3)
    rbs = (rb0, rb1, rb2, rb3)
    gsem = (gs0, gs1, gs2, gs3)
    ssem = (ss0, ss1, ss2, ss3)
    msem = (ms0, ms1, ms2, ms3)
    rsem = (rs0, rs1, rs2, rs3)

    for r in range(ZB):
        for j in range(HALF // LANES):
            zbuf_v[r, pl.ds(j * LANES, LANES)] = jnp.zeros((LANES,),
                                                           jnp.float32)

    def zcopy(k, carry):
        pltpu.sync_copy(zbuf_v, acc_sh.at[pl.ds(s * R0 + k * ZB, ZB)])
        return carry

    lax.fori_loop(0, R0 // ZB, zcopy, 0)

    @pl.when(s == NS - 1)
    def _():
        pltpu.sync_copy(zbuf_v, acc_sh.at[pl.ds(R0 * NS, TAIL)])

    plsc.subcore_barrier()

    def issue_meta(slot, hb, b):
        pltpu.async_copy(cols_hbm.at[pl.ds(hb + b * BB, BB)], colsb[slot],
                         msem[slot])
        pltpu.async_copy(vals_hbm.at[pl.ds(hb + b * BB, BB)], valsb[slot],
                         msem[slot])

    def wait_meta(slot, hb):
        pltpu.make_async_copy(cols_hbm.at[pl.ds(hb, BB)], colsb[slot],
                              msem[slot]).wait()
        pltpu.make_async_copy(vals_hbm.at[pl.ds(hb, BB)], valsb[slot],
                              msem[slot]).wait()

    def mkidx(slot, off_vec):
        for j in range(BB // LANES):
            sl = pl.ds(j * LANES, LANES)
            idxb[slot][sl] = colsb[slot][sl] + off_vec

    HB = BB // 2

    def issue_gather_rows(slot, hb, b):
        pltpu.async_copy(fw_hbm.at[idxb[slot].at[pl.ds(0, HB)]],
                         gath[slot].at[pl.ds(0, HB)], gsem[slot])
        pltpu.async_copy(fw_hbm.at[idxb[slot].at[pl.ds(HB, HB)]],
                         gath[slot].at[pl.ds(HB, HB)], gsem[slot])
        pltpu.async_copy(rows_hbm.at[pl.ds(hb + b * BB, HB)],
                         rbs[slot].at[0], rsem[slot])
        pltpu.async_copy(rows_hbm.at[pl.ds(hb + b * BB + HB, HB)],
                         rbs[slot].at[1], rsem[slot])

    def wait_gather(slot):
        for _ in range(2):
            pltpu.make_async_copy(fw_hbm.at[idxb[slot].at[pl.ds(0, HB)]],
                                  gath[slot].at[pl.ds(0, HB)],
                                  gsem[slot]).wait()

    def wait_rows(slot, hb):
        for _ in range(2):
            pltpu.make_async_copy(rows_hbm.at[pl.ds(hb, HB)],
                                  rbs[slot].at[0], rsem[slot]).wait()

    def wait_scatter(slot):
        for _ in range(2):
            pltpu.make_async_copy(gath[slot].at[pl.ds(0, HB)],
                                  acc_sh.at[rbs[slot].at[0]],
                                  ssem[slot]).wait()

    def issue_scatter(slot):
        pltpu.async_copy(gath[slot].at[pl.ds(0, HB)],
                         acc_sh.at[rbs[slot].at[0]], ssem[slot], add=True)
        pltpu.async_copy(gath[slot].at[pl.ds(HB, HB)],
                         acc_sh.at[rbs[slot].at[1]], ssem[slot], add=True)

    def scale(slot):
        def grp(g, carry):
            vv = valsb[slot][pl.ds(g * LANES, LANES)]
            for l in range(LANES):
                r = g * LANES + l
                v = vv[l]
                for j in range(HALF // LANES):
                    sl = pl.ds(j * LANES, LANES)
                    gath[slot][r, sl] = gath[slot][r, sl] * v
            return carry

        lax.fori_loop(0, BB // LANES, grp, 0)

    def hop_body(h, carry):
        hb = h * E + s * EPT
        off_vec = jnp.zeros((LANES,), jnp.int32) + (h * NC + c) * N

        issue_meta(0, hb, jnp.int32(0))
        issue_meta(1, hb, jnp.int32(1))
        wait_meta(0, hb)
        mkidx(0, off_vec)
        issue_gather_rows(0, hb, jnp.int32(0))
        wait_meta(1, hb)
        mkidx(1, off_vec)
        issue_gather_rows(1, hb, jnp.int32(1))
        issue_meta(2, hb, jnp.int32(2))
        issue_meta(3, hb, jnp.int32(3))

        NQ = NBATCH // NSLOT

        def step(p, k):
            b = NSLOT * p + k
            g2 = (k + 2) % NSLOT
            wait_gather(k)
            if k < 2:
                @pl.when(p > 0)
                def _():
                    wait_scatter(g2)
            else:
                wait_scatter(g2)
            if k == NSLOT - 1:
                @pl.when(p < NQ - 1)
                def _():
                    wait_meta(g2, hb)
                    mkidx(g2, off_vec)
                    issue_gather_rows(g2, hb, b + 2)
            else:
                wait_meta(g2, hb)
                mkidx(g2, off_vec)
                issue_gather_rows(g2, hb, b + 2)
            scale(k)
            if k == 0:
                issue_meta(k, hb, b + NSLOT)
            else:
                @pl.when(p < NQ - 1)
                def _():
                    issue_meta(k, hb, b + NSLOT)
            wait_rows(k, hb)
            issue_scatter(k)

        def quad(p, cy):
            for k in range(NSLOT):
                step(p, k)
            return cy

        lax.fori_loop(0, NQ, quad, 0)

        wait_gather(0)
        scale(0)
        wait_rows(0, hb)
        issue_scatter(0)
        wait_scatter(2)
        wait_scatter(3)
        wait_scatter(0)
        return carry

    lax.fori_loop(0, NH, hop_body, 0)

    plsc.subcore_barrier()
    pltpu.sync_copy(acc_sh.at[pl.ds(s * R0, R0)],
                    out_hbm.at[c, pl.ds(s * R0, R0)])

    @pl.when(s == NS - 1)
    def _():
        pltpu.sync_copy(acc_sh.at[pl.ds(R0 * NS, TAIL)],
                        out_hbm.at[c, pl.ds(R0 * NS, TAIL)])


_sc_call = functools.partial(
    pl.kernel,
    mesh=plsc.VectorSubcoreMesh(core_axis_name="c", subcore_axis_name="s"),
    out_type=jax.ShapeDtypeStruct((NC, N, HALF), jnp.float32),
    scratch_types=(
        [pltpu.VMEM((BB, HALF), jnp.float32) for _ in range(NSLOT)]
        + [pltpu.VMEM((BB,), jnp.int32) for _ in range(NSLOT)]
        + [pltpu.VMEM((BB,), jnp.float32) for _ in range(NSLOT)]
        + [pltpu.VMEM((BB,), jnp.int32) for _ in range(NSLOT)]
        + [pltpu.VMEM((2, BB // 2), jnp.int32) for _ in range(NSLOT)]
        + [pltpu.VMEM((ZB, HALF), jnp.float32)]
        + [pltpu.VMEM_SHARED((N, HALF), jnp.float32)]
        + [pltpu.SemaphoreType.DMA for _ in range(4 * NSLOT)]
    ),
)(_sc_spmm_body)


def _out_body(alpha_ref, st_ref, le_ref, g_ref, o_ref):
    a = alpha_ref[0, 0]
    lg = lax.dot_general(le_ref[...], g_ref[...], (((1,), (0,)), ((), ())),
                         preferred_element_type=jnp.float32)
    o_ref[:, :HALF] = jnp.maximum(
        st_ref[0].astype(jnp.float32) + a * lg[:, :HALF], 0.0)
    o_ref[:, HALF:] = jnp.maximum(
        st_ref[1].astype(jnp.float32) + a * lg[:, HALF:], 0.0)


_out_call = pl.pallas_call(
    _out_body,
    grid=(GRID,),
    in_specs=[
        pl.BlockSpec(memory_space=pltpu.SMEM),
        pl.BlockSpec((NC, RB, HALF), lambda i: (0, i, 0)),
        pl.BlockSpec((RB, OUT), lambda i: (i, 0)),
        pl.BlockSpec((OUT, OUT), lambda i: (0, 0)),
    ],
    out_specs=pl.BlockSpec((RB, OUT), lambda i: (i, 0)),
    out_shape=jax.ShapeDtypeStruct((N, OUT), jnp.float32),
)


def kernel(X, adj_edge_index, adj_values, W_embed, W_kernels, alpha):
    w_cat = jnp.concatenate(
        [W_embed] + [W_kernels[h] for h in range(NH)], axis=1)
    le, fw, g = _mm_call(X, w_cat)
    fw_flat = fw.reshape(NH * NC * N, HALF)
    rows_flat = adj_edge_index[:, 0, :].reshape(-1)
    cols_flat = adj_edge_index[:, 1, :].reshape(-1)
    vals_flat = adj_values.reshape(-1)
    structural = _sc_call(fw_flat, rows_flat, cols_flat, vals_flat)
    alpha_arr = jnp.reshape(alpha, (1, 1)).astype(jnp.float32)
    return _out_call(alpha_arr, structural, le, g)


# speedup vs baseline: 1.3088x; 1.0091x over previous
import functools

import jax
import jax.numpy as jnp
from jax import lax
from jax.experimental import pallas as pl
from jax.experimental.pallas import tpu as pltpu
from jax.experimental.pallas import tpu_sc as plsc

N = 10000
F = 256
OUT = 256
NH = 4
E = 160000
HALF = 128
LANES = 16

RB = 400
GRID = N // RB

NC = 2
NS = 16
TPB = NH * E // NS
BB = 80
NBATCH = TPB // BB
NSLOT = 4
R0 = 624
TAIL = N - R0 * NS
ZB = 16


def _mm_body(x_ref, w_ref, le_ref, fw_ref, g_ref, acc_ref):
    i = pl.program_id(0)
    p = lax.dot_general(x_ref[...], w_ref[...], (((1,), (0,)), ((), ())),
                        preferred_element_type=jnp.float32)
    le = p[:, :OUT]
    le_ref[...] = le
    for h in range(NH):
        base = OUT + h * OUT
        for s in range(NC):
            fw_ref[h, s] = p[:, base + s * HALF:base + (s + 1) * HALF]
    fwsum = (p[:, OUT:2 * OUT] + p[:, 2 * OUT:3 * OUT]
             + p[:, 3 * OUT:4 * OUT] + p[:, 4 * OUT:5 * OUT])
    contrib = lax.dot_general(le, fwsum, (((0,), (0,)), ((), ())),
                              preferred_element_type=jnp.float32)

    @pl.when(i == 0)
    def _():
        acc_ref[...] = jnp.zeros_like(acc_ref)

    acc_ref[...] += contrib

    @pl.when(i == pl.num_programs(0) - 1)
    def _():
        g_ref[...] = acc_ref[...]


_mm_call = pl.pallas_call(
    _mm_body,
    grid=(GRID,),
    in_specs=[
        pl.BlockSpec((RB, F), lambda i: (i, 0)),
        pl.BlockSpec((F, (NH + 1) * OUT), lambda i: (0, 0)),
    ],
    out_specs=[
        pl.BlockSpec((RB, OUT), lambda i: (i, 0)),
        pl.BlockSpec((NH, NC, RB, HALF), lambda i: (0, 0, i, 0)),
        pl.BlockSpec((OUT, OUT), lambda i: (0, 0)),
    ],
    out_shape=[
        jax.ShapeDtypeStruct((N, OUT), jnp.float32),
        jax.ShapeDtypeStruct((NH, NC, N, HALF), jnp.float32),
        jax.ShapeDtypeStruct((OUT, OUT), jnp.float32),
    ],
    scratch_shapes=[pltpu.VMEM((OUT, OUT), jnp.float32)],
)


def _sc_spmm_body(fw_hbm, rows_hbm, cols_hbm, vals_hbm, out_hbm,
                  g0, g1, g2, g3, cb0, cb1, cb2, cb3, vb0, vb1, vb2, vb3,
                  ib0, ib1, ib2, ib3, rb0, rb1, rb2, rb3, zbuf_v,
                  acc_sh, gs0, gs1, gs2, gs3, ss0, ss1, ss2, ss3,
                  ms0, ms1, ms2, ms3, rs0, rs1, rs2, rs3):
    c = lax.axis_index("c")
    s = lax.axis_index("s")
    gath = (g0, g1, g2, g3)
    colsb = (cb0, cb1, cb2, cb3)
    valsb = (vb0, vb1, vb2, vb3)
    idxb = (ib0, ib1, ib2, ib3)
    rbs = (rb0, rb1, rb2, rb3)
    gsem = (gs0, gs1, gs2, gs3)
    ssem = (ss0, ss1, ss2, ss3)
    msem = (ms0, ms1, ms2, ms3)
    rsem = (rs0, rs1, rs2, rs3)

    for r in range(ZB):
        for j in range(HALF // LANES):
            zbuf_v[r, pl.ds(j * LANES, LANES)] = jnp.zeros((LANES,),
                                                           jnp.float32)

    def zcopy(k, carry):
        pltpu.sync_copy(zbuf_v, acc_sh.at[pl.ds(s * R0 + k * ZB, ZB)])
        return carry

    lax.fori_loop(0, R0 // ZB, zcopy, 0)

    @pl.when(s == NS - 1)
    def _():
        pltpu.sync_copy(zbuf_v, acc_sh.at[pl.ds(R0 * NS, TAIL)])

    def issue_meta(slot, hb, b):
        pltpu.async_copy(cols_hbm.at[pl.ds(hb + b * BB, BB)], colsb[slot],
                         msem[slot])
        pltpu.async_copy(vals_hbm.at[pl.ds(hb + b * BB, BB)], valsb[slot],
                         msem[slot])

    def wait_meta(slot, hb):
        pltpu.make_async_copy(cols_hbm.at[pl.ds(hb, BB)], colsb[slot],
                              msem[slot]).wait()
        pltpu.make_async_copy(vals_hbm.at[pl.ds(hb, BB)], valsb[slot],
                              msem[slot]).wait()

    def mkidx(slot, off_vec):
        for j in range(BB // LANES):
            sl = pl.ds(j * LANES, LANES)
            idxb[slot][sl] = colsb[slot][sl] + off_vec

    HB = BB // 2

    def issue_gather_rows(slot, hb, b):
        pltpu.async_copy(fw_hbm.at[idxb[slot].at[pl.ds(0, HB)]],
                         gath[slot].at[pl.ds(0, HB)], gsem[slot])
        pltpu.async_copy(fw_hbm.at[idxb[slot].at[pl.ds(HB, HB)]],
                         gath[slot].at[pl.ds(HB, HB)], gsem[slot])
        pltpu.async_copy(rows_hbm.at[pl.ds(hb + b * BB, HB)],
                         rbs[slot].at[0], rsem[slot])
        pltpu.async_copy(rows_hbm.at[pl.ds(hb + b * BB + HB, HB)],
                         rbs[slot].at[1], rsem[slot])

    def wait_gather(slot):
        for _ in range(2):
            pltpu.make_async_copy(fw_hbm.at[idxb[slot].at[pl.ds(0, HB)]],
                                  gath[slot].at[pl.ds(0, HB)],
                                  gsem[slot]).wait()

    def wait_rows(slot, hb):
        for _ in range(2):
            pltpu.make_async_copy(rows_hbm.at[pl.ds(hb, HB)],
                                  rbs[slot].at[0], rsem[slot]).wait()

    def wait_scatter(slot):
        for _ in range(2):
            pltpu.make_async_copy(gath[slot].at[pl.ds(0, HB)],
                                  acc_sh.at[rbs[slot].at[0]],
                                  ssem[slot]).wait()

    def issue_scatter(slot):
        pltpu.async_copy(gath[slot].at[pl.ds(0, HB)],
                         acc_sh.at[rbs[slot].at[0]], ssem[slot], add=True)
        pltpu.async_copy(gath[slot].at[pl.ds(HB, HB)],
                         acc_sh.at[rbs[slot].at[1]], ssem[slot], add=True)

    def scale(slot):
        def grp(g, carry):
            vv = valsb[slot][pl.ds(g * LANES, LANES)]
            for l in range(LANES):
                r = g * LANES + l
                v = vv[l]
                for j in range(HALF // LANES):
                    sl = pl.ds(j * LANES, LANES)
                    gath[slot][r, sl] = gath[slot][r, sl] * v
            return carry

        lax.fori_loop(0, BB // LANES, grp, 0)

    hb = s * TPB
    off_vec = jnp.zeros((LANES,), jnp.int32) + ((s // (NS // NH)) * NC + c) * N

    issue_meta(0, hb, jnp.int32(0))
    issue_meta(1, hb, jnp.int32(1))
    wait_meta(0, hb)
    mkidx(0, off_vec)
    issue_gather_rows(0, hb, jnp.int32(0))
    wait_meta(1, hb)
    mkidx(1, off_vec)
    issue_gather_rows(1, hb, jnp.int32(1))
    issue_meta(2, hb, jnp.int32(2))
    issue_meta(3, hb, jnp.int32(3))

    plsc.subcore_barrier()

    NQ = NBATCH // NSLOT

    def step(p, k):
        b = NSLOT * p + k
        g2 = (k + 2) % NSLOT
        wait_gather(k)
        if k < 2:
            @pl.when(p > 0)
            def _():
                wait_scatter(g2)
        else:
            wait_scatter(g2)
        if k < 2:
            wait_meta(g2, hb)
            mkidx(g2, off_vec)
            issue_gather_rows(g2, hb, b + 2)
        else:
            @pl.when(p < NQ - 1)
            def _():
                wait_meta(g2, hb)
                mkidx(g2, off_vec)
                issue_gather_rows(g2, hb, b + 2)
        scale(k)

        @pl.when(p < NQ - 1)
        def _():
            issue_meta(k, hb, b + NSLOT)

        wait_rows(k, hb)
        issue_scatter(k)

    def quad(p, cy):
        for k in range(NSLOT):
            step(p, k)
        return cy

    lax.fori_loop(0, NQ, quad, 0)

    wait_scatter(2)
    wait_scatter(3)

    plsc.subcore_barrier()
    pltpu.sync_copy(acc_sh.at[pl.ds(s * R0, R0)],
                    out_hbm.at[c, pl.ds(s * R0, R0)])

    @pl.when(s == NS - 1)
    def _():
        pltpu.sync_copy(acc_sh.at[pl.ds(R0 * NS, TAIL)],
                        out_hbm.at[c, pl.ds(R0 * NS, TAIL)])


_sc_call = functools.partial(
    pl.kernel,
    mesh=plsc.VectorSubcoreMesh(core_axis_name="c", subcore_axis_name="s"),
    out_type=jax.ShapeDtypeStruct((NC, N, HALF), jnp.float32),
    scratch_types=(
        [pltpu.VMEM((BB, HALF), jnp.float32) for _ in range(NSLOT)]
        + [pltpu.VMEM((BB,), jnp.int32) for _ in range(NSLOT)]
        + [pltpu.VMEM((BB,), jnp.float32) for _ in range(NSLOT)]
        + [pltpu.VMEM((BB,), jnp.int32) for _ in range(NSLOT)]
        + [pltpu.VMEM((2, BB // 2), jnp.int32) for _ in range(NSLOT)]
        + [pltpu.VMEM((ZB, HALF), jnp.float32)]
        + [pltpu.VMEM_SHARED((N, HALF), jnp.float32)]
        + [pltpu.SemaphoreType.DMA for _ in range(4 * NSLOT)]
    ),
)(_sc_spmm_body)


def _out_body(alpha_ref, st_ref, le_ref, g_ref, o_ref):
    a = alpha_ref[0, 0]
    lg = lax.dot_general(le_ref[...], g_ref[...], (((1,), (0,)), ((), ())),
                         preferred_element_type=jnp.float32)
    o_ref[:, :HALF] = jnp.maximum(
        st_ref[0].astype(jnp.float32) + a * lg[:, :HALF], 0.0)
    o_ref[:, HALF:] = jnp.maximum(
        st_ref[1].astype(jnp.float32) + a * lg[:, HALF:], 0.0)


_out_call = pl.pallas_call(
    _out_body,
    grid=(GRID,),
    in_specs=[
        pl.BlockSpec(memory_space=pltpu.SMEM),
        pl.BlockSpec((NC, RB, HALF), lambda i: (0, i, 0)),
        pl.BlockSpec((RB, OUT), lambda i: (i, 0)),
        pl.BlockSpec((OUT, OUT), lambda i: (0, 0)),
    ],
    out_specs=pl.BlockSpec((RB, OUT), lambda i: (i, 0)),
    out_shape=jax.ShapeDtypeStruct((N, OUT), jnp.float32),
)


def kernel(X, adj_edge_index, adj_values, W_embed, W_kernels, alpha):
    w_cat = jnp.concatenate(
        [W_embed] + [W_kernels[h] for h in range(NH)], axis=1)
    le, fw, g = _mm_call(X, w_cat)
    fw_flat = fw.reshape(NH * NC * N, HALF)
    rows_flat = adj_edge_index[:, 0, :].reshape(-1)
    cols_flat = adj_edge_index[:, 1, :].reshape(-1)
    vals_flat = adj_values.reshape(-1)
    structural = _sc_call(fw_flat, rows_flat, cols_flat, vals_flat)
    alpha_arr = jnp.reshape(alpha, (1, 1)).astype(jnp.float32)
    return _out_call(alpha_arr, structural, le, g)


# speedup vs baseline: 1.3100x; 1.0009x over previous
import functools

import jax
import jax.numpy as jnp
from jax import lax
from jax.experimental import pallas as pl
from jax.experimental.pallas import tpu as pltpu
from jax.experimental.pallas import tpu_sc as plsc

N = 10000
F = 256
OUT = 256
NH = 4
E = 160000
HALF = 128
LANES = 16

RB = 400
GRID = N // RB

NC = 2
NS = 16
TPB = NH * E // NS
BB = 80
NBATCH = TPB // BB
NSLOT = 4
R0 = 624
TAIL = N - R0 * NS
ZB = 16


def _mm_body(x_ref, w_ref, le_ref, fw_ref, g_ref, acc_ref):
    i = pl.program_id(0)
    p = lax.dot_general(x_ref[...], w_ref[...], (((1,), (0,)), ((), ())),
                        preferred_element_type=jnp.float32)
    le = p[:, :OUT]
    le_ref[...] = le
    for h in range(NH):
        base = OUT + h * OUT
        for s in range(NC):
            fw_ref[h, s] = p[:, base + s * HALF:base + (s + 1) * HALF]
    fwsum = (p[:, OUT:2 * OUT] + p[:, 2 * OUT:3 * OUT]
             + p[:, 3 * OUT:4 * OUT] + p[:, 4 * OUT:5 * OUT])
    contrib = lax.dot_general(le, fwsum, (((0,), (0,)), ((), ())),
                              preferred_element_type=jnp.float32)

    @pl.when(i == 0)
    def _():
        acc_ref[...] = jnp.zeros_like(acc_ref)

    acc_ref[...] += contrib

    @pl.when(i == pl.num_programs(0) - 1)
    def _():
        g_ref[...] = acc_ref[...]


_mm_call = pl.pallas_call(
    _mm_body,
    grid=(GRID,),
    in_specs=[
        pl.BlockSpec((RB, F), lambda i: (i, 0)),
        pl.BlockSpec((F, (NH + 1) * OUT), lambda i: (0, 0)),
    ],
    out_specs=[
        pl.BlockSpec((RB, OUT), lambda i: (i, 0)),
        pl.BlockSpec((NH, NC, RB, HALF), lambda i: (0, 0, i, 0)),
        pl.BlockSpec((OUT, OUT), lambda i: (0, 0)),
    ],
    out_shape=[
        jax.ShapeDtypeStruct((N, OUT), jnp.float32),
        jax.ShapeDtypeStruct((NH, NC, N, HALF), jnp.float32),
        jax.ShapeDtypeStruct((OUT, OUT), jnp.float32),
    ],
    scratch_shapes=[pltpu.VMEM((OUT, OUT), jnp.float32)],
)


def _sc_spmm_body(fw_hbm, rows_hbm, cols_hbm, vals_hbm, out_hbm,
                  g0, g1, g2, g3, cb0, cb1, cb2, cb3, vb0, vb1, vb2, vb3,
                  ib0, ib1, ib2, ib3, rb0, rb1, rb2, rb3, zbuf_v,
                  acc_sh, gs0, gs1, gs2, gs3, ss0, ss1, ss2, ss3,
                  ms0, ms1, ms2, ms3, rs0, rs1, rs2, rs3):
    c = lax.axis_index("c")
    s = lax.axis_index("s")
    gath = (g0, g1, g2, g3)
    colsb = (cb0, cb1, cb2, cb3)
    valsb = (vb0, vb1, vb2, vb3)
    idxb = (ib0, ib1, ib2, ib3)
    rbs = (rb0, rb1, rb2, rb3)
    gsem = (gs0, gs1, gs2, gs3)
    ssem = (ss0, ss1, ss2, ss3)
    msem = (ms0, ms1, ms2, ms3)
    rsem = (rs0, rs1, rs2, rs3)

    for r in range(ZB):
        for j in range(HALF // LANES):
            zbuf_v[r, pl.ds(j * LANES, LANES)] = jnp.zeros((LANES,),
                                                           jnp.float32)

    def zcopy(k, carry):
        pltpu.sync_copy(zbuf_v, acc_sh.at[pl.ds(s * R0 + k * ZB, ZB)])
        return carry

    lax.fori_loop(0, R0 // ZB, zcopy, 0)

    @pl.when(s == NS - 1)
    def _():
        pltpu.sync_copy(zbuf_v, acc_sh.at[pl.ds(R0 * NS, TAIL)])

    def issue_meta(slot, hb, b):
        pltpu.async_copy(cols_hbm.at[pl.ds(hb + b * BB, BB)], colsb[slot],
                         msem[slot])
        pltpu.async_copy(vals_hbm.at[pl.ds(hb + b * BB, BB)], valsb[slot],
                         msem[slot])

    def wait_meta(slot, hb):
        pltpu.make_async_copy(cols_hbm.at[pl.ds(hb, BB)], colsb[slot],
                              msem[slot]).wait()
        pltpu.make_async_copy(vals_hbm.at[pl.ds(hb, BB)], valsb[slot],
                              msem[slot]).wait()

    def mkidx(slot, off_vec):
        for j in range(BB // LANES):
            sl = pl.ds(j * LANES, LANES)
            idxb[slot][sl] = colsb[slot][sl] + off_vec

    HB = BB // 2

    def issue_gather_rows(slot, hb, b):
        pltpu.async_copy(fw_hbm.at[idxb[slot].at[pl.ds(0, HB)]],
                         gath[slot].at[pl.ds(0, HB)], gsem[slot])
        pltpu.async_copy(fw_hbm.at[idxb[slot].at[pl.ds(HB, HB)]],
                         gath[slot].at[pl.ds(HB, HB)], gsem[slot])
        pltpu.async_copy(rows_hbm.at[pl.ds(hb + b * BB, HB)],
                         rbs[slot].at[0], rsem[slot])
        pltpu.async_copy(rows_hbm.at[pl.ds(hb + b * BB + HB, HB)],
                         rbs[slot].at[1], rsem[slot])

    def wait_gather(slot):
        for _ in range(2):
            pltpu.make_async_copy(fw_hbm.at[idxb[slot].at[pl.ds(0, HB)]],
                                  gath[slot].at[pl.ds(0, HB)],
                                  gsem[slot]).wait()

    def wait_rows(slot, hb):
        for _ in range(2):
            pltpu.make_async_copy(rows_hbm.at[pl.ds(hb, HB)],
                                  rbs[slot].at[0], rsem[slot]).wait()

    def wait_scatter(slot):
        for _ in range(2):
            pltpu.make_async_copy(gath[slot].at[pl.ds(0, HB)],
                                  acc_sh.at[rbs[slot].at[0]],
                                  ssem[slot]).wait()

    def issue_scatter(slot):
        pltpu.async_copy(gath[slot].at[pl.ds(0, HB)],
                         acc_sh.at[rbs[slot].at[0]], ssem[slot], add=True)
        pltpu.async_copy(gath[slot].at[pl.ds(HB, HB)],
                         acc_sh.at[rbs[slot].at[1]], ssem[slot], add=True)

    def scale(slot):
        def grp(g, carry):
            vv = valsb[slot][pl.ds(g * LANES, LANES)]
            for l in range(LANES):
                r = g * LANES + l
                v = vv[l]
                for j in range(HALF // LANES):
                    sl = pl.ds(j * LANES, LANES)
                    gath[slot][r, sl] = gath[slot][r, sl] * v
            return carry

        lax.fori_loop(0, BB // LANES, grp, 0)

    hb = s * TPB
    off_vec = jnp.zeros((LANES,), jnp.int32) + ((s // (NS // NH)) * NC + c) * N

    issue_meta(0, hb, jnp.int32(0))
    issue_meta(1, hb, jnp.int32(1))
    wait_meta(0, hb)
    mkidx(0, off_vec)
    issue_gather_rows(0, hb, jnp.int32(0))
    wait_meta(1, hb)
    mkidx(1, off_vec)
    issue_gather_rows(1, hb, jnp.int32(1))
    issue_meta(2, hb, jnp.int32(2))
    issue_meta(3, hb, jnp.int32(3))

    plsc.subcore_barrier()

    NQ = NBATCH // NSLOT

    def step(p, k):
        b = NSLOT * p + k
        g2 = (k + 2) % NSLOT
        wait_gather(k)
        if k < 2:
            @pl.when(p > 0)
            def _():
                wait_scatter(g2)
        else:
            wait_scatter(g2)
        if k < 2:
            wait_meta(g2, hb)
            mkidx(g2, off_vec)
            issue_gather_rows(g2, hb, b + 2)
        else:
            @pl.when(p < NQ - 1)
            def _():
                wait_meta(g2, hb)
                mkidx(g2, off_vec)
                issue_gather_rows(g2, hb, b + 2)
        scale(k)

        @pl.when(p < NQ - 1)
        def _():
            issue_meta(k, hb, b + NSLOT)

        wait_rows(k, hb)
        issue_scatter(k)

    def quad(p, cy):
        for k in range(NSLOT):
            step(p, k)
        return cy

    lax.fori_loop(0, NQ, quad, 0)

    wait_scatter(2)
    wait_scatter(3)

    plsc.subcore_barrier()
    pltpu.sync_copy(acc_sh.at[pl.ds(s * R0, R0)],
                    out_hbm.at[c, pl.ds(s * R0, R0)])

    @pl.when(s == NS - 1)
    def _():
        pltpu.sync_copy(acc_sh.at[pl.ds(R0 * NS, TAIL)],
                        out_hbm.at[c, pl.ds(R0 * NS, TAIL)])


_sc_call = functools.partial(
    pl.kernel,
    mesh=plsc.VectorSubcoreMesh(core_axis_name="c", subcore_axis_name="s"),
    out_type=jax.ShapeDtypeStruct((NC, N, HALF), jnp.float32),
    scratch_types=(
        [pltpu.VMEM((BB, HALF), jnp.float32) for _ in range(NSLOT)]
        + [pltpu.VMEM((BB,), jnp.int32) for _ in range(NSLOT)]
        + [pltpu.VMEM((BB,), jnp.float32) for _ in range(NSLOT)]
        + [pltpu.VMEM((BB,), jnp.int32) for _ in range(NSLOT)]
        + [pltpu.VMEM((2, BB // 2), jnp.int32) for _ in range(NSLOT)]
        + [pltpu.VMEM((ZB, HALF), jnp.float32)]
        + [pltpu.VMEM_SHARED((N, HALF), jnp.float32)]
        + [pltpu.SemaphoreType.DMA for _ in range(4 * NSLOT)]
    ),
)(_sc_spmm_body)


def _learned_body(alpha_ref, le_ref, g_ref, o_ref):
    a = alpha_ref[0, 0]
    lg = lax.dot_general(le_ref[...], g_ref[...], (((1,), (0,)), ((), ())),
                         preferred_element_type=jnp.float32)
    o_ref[...] = a * lg


_learned_call = pl.pallas_call(
    _learned_body,
    grid=(GRID,),
    in_specs=[
        pl.BlockSpec(memory_space=pltpu.SMEM),
        pl.BlockSpec((RB, OUT), lambda i: (i, 0)),
        pl.BlockSpec((OUT, OUT), lambda i: (0, 0)),
    ],
    out_specs=pl.BlockSpec((RB, OUT), lambda i: (i, 0)),
    out_shape=jax.ShapeDtypeStruct((N, OUT), jnp.float32),
)


def _out_body(st_ref, lg_ref, o_ref):
    lg = lg_ref[...]
    o_ref[:, :HALF] = jnp.maximum(st_ref[0] + lg[:, :HALF], 0.0)
    o_ref[:, HALF:] = jnp.maximum(st_ref[1] + lg[:, HALF:], 0.0)


_out_call = pl.pallas_call(
    _out_body,
    grid=(GRID,),
    in_specs=[
        pl.BlockSpec((NC, RB, HALF), lambda i: (0, i, 0)),
        pl.BlockSpec((RB, OUT), lambda i: (i, 0)),
    ],
    out_specs=pl.BlockSpec((RB, OUT), lambda i: (i, 0)),
    out_shape=jax.ShapeDtypeStruct((N, OUT), jnp.float32),
)


def kernel(X, adj_edge_index, adj_values, W_embed, W_kernels, alpha):
    w_cat = jnp.concatenate(
        [W_embed] + [W_kernels[h] for h in range(NH)], axis=1)
    le, fw, g = _mm_call(X, w_cat)
    fw_flat = fw.reshape(NH * NC * N, HALF)
    rows_flat = adj_edge_index[:, 0, :].reshape(-1)
    cols_flat = adj_edge_index[:, 1, :].reshape(-1)
    vals_flat = adj_values.reshape(-1)
    structural = _sc_call(fw_flat, rows_flat, cols_flat, vals_flat)
    alpha_arr = jnp.reshape(alpha, (1, 1)).astype(jnp.float32)
    learned = _learned_call(alpha_arr, le, g)
    return _out_call(structural, learned)


# speedup vs baseline: 1.3108x; 1.0006x over previous
import functools

import jax
import jax.numpy as jnp
from jax import lax
from jax.experimental import pallas as pl
from jax.experimental.pallas import tpu as pltpu
from jax.experimental.pallas import tpu_sc as plsc

N = 10000
F = 256
OUT = 256
NH = 4
E = 160000
HALF = 128
LANES = 16

RB = 400
GRID = N // RB

NC = 2
NS = 16
TPB = NH * E // NS
BB = 80
NBATCH = TPB // BB
NSLOT = 4
R0 = 624
TAIL = N - R0 * NS
ZB = 16


def _mm_body(x_ref, w_ref, le_ref, fw_ref, g_ref, acc_ref):
    i = pl.program_id(0)
    p = lax.dot_general(x_ref[...].astype(jnp.bfloat16),
                        w_ref[...].astype(jnp.bfloat16),
                        (((1,), (0,)), ((), ())),
                        preferred_element_type=jnp.float32)
    le = p[:, :OUT]
    le_ref[...] = le
    for h in range(NH):
        base = OUT + h * OUT
        for s in range(NC):
            fw_ref[h, s] = p[:, base + s * HALF:base + (s + 1) * HALF]
    fwsum = (p[:, OUT:2 * OUT] + p[:, 2 * OUT:3 * OUT]
             + p[:, 3 * OUT:4 * OUT] + p[:, 4 * OUT:5 * OUT])
    contrib = lax.dot_general(le, fwsum, (((0,), (0,)), ((), ())),
                              preferred_element_type=jnp.float32)

    @pl.when(i == 0)
    def _():
        acc_ref[...] = jnp.zeros_like(acc_ref)

    acc_ref[...] += contrib

    @pl.when(i == pl.num_programs(0) - 1)
    def _():
        g_ref[...] = acc_ref[...]


_mm_call = pl.pallas_call(
    _mm_body,
    grid=(GRID,),
    in_specs=[
        pl.BlockSpec((RB, F), lambda i: (i, 0)),
        pl.BlockSpec((F, (NH + 1) * OUT), lambda i: (0, 0)),
    ],
    out_specs=[
        pl.BlockSpec((RB, OUT), lambda i: (i, 0)),
        pl.BlockSpec((NH, NC, RB, HALF), lambda i: (0, 0, i, 0)),
        pl.BlockSpec((OUT, OUT), lambda i: (0, 0)),
    ],
    out_shape=[
        jax.ShapeDtypeStruct((N, OUT), jnp.float32),
        jax.ShapeDtypeStruct((NH, NC, N, HALF), jnp.float32),
        jax.ShapeDtypeStruct((OUT, OUT), jnp.float32),
    ],
    scratch_shapes=[pltpu.VMEM((OUT, OUT), jnp.float32)],
)


def _sc_spmm_body(fw_hbm, rows_hbm, cols_hbm, vals_hbm, out_hbm,
                  g0, g1, g2, g3, cb0, cb1, cb2, cb3, vb0, vb1, vb2, vb3,
                  ib0, ib1, ib2, ib3, rb0, rb1, rb2, rb3, zbuf_v,
                  acc_sh, gs0, gs1, gs2, gs3, ss0, ss1, ss2, ss3,
                  ms0, ms1, ms2, ms3, rs0, rs1, rs2, rs3):
    c = lax.axis_index("c")
    s = lax.axis_index("s")
    gath = (g0, g1, g2, g3)
    colsb = (cb0, cb1, cb2, cb3)
    valsb = (vb0, vb1, vb2, vb3)
    idxb = (ib0, ib1, ib2, ib3)
    rbs = (rb0, rb1, rb2, rb3)
    gsem = (gs0, gs1, gs2, gs3)
    ssem = (ss0, ss1, ss2, ss3)
    msem = (ms0, ms1, ms2, ms3)
    rsem = (rs0, rs1, rs2, rs3)

    for r in range(ZB):
        for j in range(HALF // LANES):
            zbuf_v[r, pl.ds(j * LANES, LANES)] = jnp.zeros((LANES,),
                                                           jnp.float32)

    def zcopy(k, carry):
        pltpu.sync_copy(zbuf_v, acc_sh.at[pl.ds(s * R0 + k * ZB, ZB)])
        return carry

    lax.fori_loop(0, R0 // ZB, zcopy, 0)

    @pl.when(s == NS - 1)
    def _():
        pltpu.sync_copy(zbuf_v, acc_sh.at[pl.ds(R0 * NS, TAIL)])

    def issue_meta(slot, hb, b):
        pltpu.async_copy(cols_hbm.at[pl.ds(hb + b * BB, BB)], colsb[slot],
                         msem[slot])
        pltpu.async_copy(vals_hbm.at[pl.ds(hb + b * BB, BB)], valsb[slot],
                         msem[slot])

    def wait_meta(slot, hb):
        pltpu.make_async_copy(cols_hbm.at[pl.ds(hb, BB)], colsb[slot],
                              msem[slot]).wait()
        pltpu.make_async_copy(vals_hbm.at[pl.ds(hb, BB)], valsb[slot],
                              msem[slot]).wait()

    def mkidx(slot, off_vec):
        for j in range(BB // LANES):
            sl = pl.ds(j * LANES, LANES)
            idxb[slot][sl] = colsb[slot][sl] + off_vec

    HB = BB // 2

    def issue_gather_rows(slot, hb, b):
        pltpu.async_copy(fw_hbm.at[idxb[slot].at[pl.ds(0, HB)]],
                         gath[slot].at[pl.ds(0, HB)], gsem[slot])
        pltpu.async_copy(fw_hbm.at[idxb[slot].at[pl.ds(HB, HB)]],
                         gath[slot].at[pl.ds(HB, HB)], gsem[slot])
        pltpu.async_copy(rows_hbm.at[pl.ds(hb + b * BB, HB)],
                         rbs[slot].at[0], rsem[slot])
        pltpu.async_copy(rows_hbm.at[pl.ds(hb + b * BB + HB, HB)],
                         rbs[slot].at[1], rsem[slot])

    def wait_gather(slot):
        for _ in range(2):
            pltpu.make_async_copy(fw_hbm.at[idxb[slot].at[pl.ds(0, HB)]],
                                  gath[slot].at[pl.ds(0, HB)],
                                  gsem[slot]).wait()

    def wait_rows(slot, hb):
        for _ in range(2):
            pltpu.make_async_copy(rows_hbm.at[pl.ds(hb, HB)],
                                  rbs[slot].at[0], rsem[slot]).wait()

    def wait_scatter(slot):
        for _ in range(2):
            pltpu.make_async_copy(gath[slot].at[pl.ds(0, HB)],
                                  acc_sh.at[rbs[slot].at[0]],
                                  ssem[slot]).wait()

    def issue_scatter(slot):
        pltpu.async_copy(gath[slot].at[pl.ds(0, HB)],
                         acc_sh.at[rbs[slot].at[0]], ssem[slot], add=True)
        pltpu.async_copy(gath[slot].at[pl.ds(HB, HB)],
                         acc_sh.at[rbs[slot].at[1]], ssem[slot], add=True)

    def scale(slot):
        def grp(g, carry):
            vv = valsb[slot][pl.ds(g * LANES, LANES)]
            for l in range(LANES):
                r = g * LANES + l
                v = vv[l]
                for j in range(HALF // LANES):
                    sl = pl.ds(j * LANES, LANES)
                    gath[slot][r, sl] = gath[slot][r, sl] * v
            return carry

        lax.fori_loop(0, BB // LANES, grp, 0)

    hb = s * TPB
    off_vec = jnp.zeros((LANES,), jnp.int32) + ((s // (NS // NH)) * NC + c) * N

    issue_meta(0, hb, jnp.int32(0))
    issue_meta(1, hb, jnp.int32(1))
    wait_meta(0, hb)
    mkidx(0, off_vec)
    issue_gather_rows(0, hb, jnp.int32(0))
    wait_meta(1, hb)
    mkidx(1, off_vec)
    issue_gather_rows(1, hb, jnp.int32(1))
    issue_meta(2, hb, jnp.int32(2))
    issue_meta(3, hb, jnp.int32(3))

    plsc.subcore_barrier()

    NQ = NBATCH // NSLOT

    def step(p, k):
        b = NSLOT * p + k
        g2 = (k + 2) % NSLOT
        wait_gather(k)
        if k < 2:
            @pl.when(p > 0)
            def _():
                wait_scatter(g2)
        else:
            wait_scatter(g2)
        if k < 2:
            wait_meta(g2, hb)
            mkidx(g2, off_vec)
            issue_gather_rows(g2, hb, b + 2)
        else:
            @pl.when(p < NQ - 1)
            def _():
                wait_meta(g2, hb)
                mkidx(g2, off_vec)
                issue_gather_rows(g2, hb, b + 2)
        scale(k)

        @pl.when(p < NQ - 1)
        def _():
            issue_meta(k, hb, b + NSLOT)

        wait_rows(k, hb)
        issue_scatter(k)

    def quad(p, cy):
        for k in range(NSLOT):
            step(p, k)
        return cy

    lax.fori_loop(0, NQ, quad, 0)

    wait_scatter(2)
    wait_scatter(3)

    plsc.subcore_barrier()
    pltpu.sync_copy(acc_sh.at[pl.ds(s * R0, R0)],
                    out_hbm.at[c, pl.ds(s * R0, R0)])

    @pl.when(s == NS - 1)
    def _():
        pltpu.sync_copy(acc_sh.at[pl.ds(R0 * NS, TAIL)],
                        out_hbm.at[c, pl.ds(R0 * NS, TAIL)])


_sc_call = functools.partial(
    pl.kernel,
    mesh=plsc.VectorSubcoreMesh(core_axis_name="c", subcore_axis_name="s"),
    out_type=jax.ShapeDtypeStruct((NC, N, HALF), jnp.float32),
    scratch_types=(
        [pltpu.VMEM((BB, HALF), jnp.float32) for _ in range(NSLOT)]
        + [pltpu.VMEM((BB,), jnp.int32) for _ in range(NSLOT)]
        + [pltpu.VMEM((BB,), jnp.float32) for _ in range(NSLOT)]
        + [pltpu.VMEM((BB,), jnp.int32) for _ in range(NSLOT)]
        + [pltpu.VMEM((2, BB // 2), jnp.int32) for _ in range(NSLOT)]
        + [pltpu.VMEM((ZB, HALF), jnp.float32)]
        + [pltpu.VMEM_SHARED((N, HALF), jnp.float32)]
        + [pltpu.SemaphoreType.DMA for _ in range(4 * NSLOT)]
    ),
)(_sc_spmm_body)


def _learned_body(alpha_ref, le_ref, g_ref, o_ref):
    a = alpha_ref[0, 0]
    lg = lax.dot_general(le_ref[...], g_ref[...], (((1,), (0,)), ((), ())),
                         preferred_element_type=jnp.float32)
    o_ref[...] = a * lg


_learned_call = pl.pallas_call(
    _learned_body,
    grid=(GRID,),
    in_specs=[
        pl.BlockSpec(memory_space=pltpu.SMEM),
        pl.BlockSpec((RB, OUT), lambda i: (i, 0)),
        pl.BlockSpec((OUT, OUT), lambda i: (0, 0)),
    ],
    out_specs=pl.BlockSpec((RB, OUT), lambda i: (i, 0)),
    out_shape=jax.ShapeDtypeStruct((N, OUT), jnp.float32),
)


def _out_body(st_ref, lg_ref, o_ref):
    lg = lg_ref[...]
    o_ref[:, :HALF] = jnp.maximum(st_ref[0] + lg[:, :HALF], 0.0)
    o_ref[:, HALF:] = jnp.maximum(st_ref[1] + lg[:, HALF:], 0.0)


_out_call = pl.pallas_call(
    _out_body,
    grid=(GRID,),
    in_specs=[
        pl.BlockSpec((NC, RB, HALF), lambda i: (0, i, 0)),
        pl.BlockSpec((RB, OUT), lambda i: (i, 0)),
    ],
    out_specs=pl.BlockSpec((RB, OUT), lambda i: (i, 0)),
    out_shape=jax.ShapeDtypeStruct((N, OUT), jnp.float32),
)


def kernel(X, adj_edge_index, adj_values, W_embed, W_kernels, alpha):
    w_cat = jnp.concatenate(
        [W_embed] + [W_kernels[h] for h in range(NH)], axis=1)
    le, fw, g = _mm_call(X, w_cat)
    fw_flat = fw.reshape(NH * NC * N, HALF)
    rows_flat = adj_edge_index[:, 0, :].reshape(-1)
    cols_flat = adj_edge_index[:, 1, :].reshape(-1)
    vals_flat = adj_values.reshape(-1)
    structural = _sc_call(fw_flat, rows_flat, cols_flat, vals_flat)
    alpha_arr = jnp.reshape(alpha, (1, 1)).astype(jnp.float32)
    learned = _learned_call(alpha_arr, le, g)
    return _out_call(structural, learned)
